# baseline (speedup 1.0000x reference)
import functools

import jax
import jax.numpy as jnp
from jax import lax
from jax.experimental import pallas as pl
from jax.experimental.pallas import tpu as pltpu
from jax.experimental.pallas import tpu_sc as plsc

_N, _E, _H, _DE, _L = 10000, 320000, 128, 16, 3
_NC, _NS = 2, 16
_NW = _NC * _NS
_CH = 64
_CHUNKS = 168
_IG = 8
_EPT = _CHUNKS * _CH
_EPAD = _EPT * _NW
_NPAD = 10112
_RPS = _NPAD // _NS
_BN = 1264
_GN = _NPAD // _BN
_BZ = 4096
_C1 = 1.5957691216057308
_C2 = _C1 * 0.044715


def _make_sc_agg():
  mesh = plsc.VectorSubcoreMesh(core_axis_name="c", subcore_axis_name="s")
  scratch = [
      pltpu.VMEM((2, _IG, _CH), jnp.int32),
      pltpu.VMEM((2, _IG, _CH), jnp.int32),
      pltpu.VMEM((2, _CH, _H), jnp.float32),
      pltpu.VMEM((3, _CH, _H), jnp.float32),
      pltpu.VMEM_SHARED((_NPAD, _H), jnp.float32),
      pltpu.SemaphoreType.DMA,
      pltpu.SemaphoreType.DMA,
      pltpu.SemaphoreType.DMA,
      pltpu.SemaphoreType.DMA,
      pltpu.SemaphoreType.DMA,
      pltpu.SemaphoreType.DMA,
      pltpu.SemaphoreType.DMA,
      pltpu.SemaphoreType.DMA,
  ]

  def body(y, z, src_i, dst_i, zer, agg_out,
           src_v, dst_v, g_v, z_v, agg_sp,
           sg0, sg1, sz0, sz1, sz2, ss0, ss1, ss2):
    sem_g = (sg0, sg1)
    sem_z = (sz0, sz1, sz2)
    sem_s = (ss0, ss1, ss2)
    c = lax.axis_index("c")
    s = lax.axis_index("s")
    w = s * _NC + c
    row0 = s * _RPS
    irow0 = w * _CHUNKS

    def stage(g):
      gp = g & 1
      pltpu.sync_copy(src_i.at[pl.ds(irow0 + g * _IG, _IG)], src_v.at[gp])
      pltpu.sync_copy(dst_i.at[pl.ds(irow0 + g * _IG, _IG)], dst_v.at[gp])

    def issue_in(j, gb, zb):
      gp = (j // _IG) & 1
      pltpu.async_copy(y.at[src_v.at[gp, j % _IG]], g_v.at[gb], sem_g[gb])
      pltpu.async_copy(z.at[pl.ds(w * _EPT + j * _CH, _CH)], z_v.at[zb],
                       sem_z[zb])

    def wait_in(gb, zb):
      pltpu.make_async_copy(y.at[src_v.at[0, 0]], g_v.at[gb],
                            sem_g[gb]).wait()
      pltpu.make_async_copy(z.at[pl.ds(0, _CH)], z_v.at[zb],
                            sem_z[zb]).wait()

    def issue_scatter(j, zb):
      gp = (j // _IG) & 1
      pltpu.async_copy(z_v.at[zb], agg_sp.at[dst_v.at[gp, j % _IG]],
                       sem_s[zb], add=True)

    def wait_scatter(zb):
      pltpu.make_async_copy(z_v.at[zb], agg_sp.at[dst_v.at[0, 0]],
                            sem_s[zb]).wait()

    pltpu.sync_copy(zer.at[pl.ds(row0, _RPS)], agg_sp.at[pl.ds(row0, _RPS)])
    plsc.subcore_barrier()

    stage(0)
    issue_in(0, 0, 0)

    def emit_chunk(j, p):
      gb, zb = p % 2, p % 3
      ngb, nzb = (p + 1) % 2, (p + 1) % 3

      @pl.when(j >= 2)
      def _():
        wait_scatter(nzb)

      @pl.when(jnp.logical_and((j + 1) % _IG == 0, j + 1 < _CHUNKS))
      def _():
        stage((j + 1) // _IG)

      @pl.when(j + 1 < _CHUNKS)
      def _():
        issue_in(j + 1, ngb, nzb)

      wait_in(gb, zb)

      def rowfn(r, cr):
        for rr in range(4):
          for cc in range(_H // 16):
            sl = pl.ds(cc * 16, 16)
            t = g_v[gb, r * 4 + rr, sl] + z_v[zb, r * 4 + rr, sl]
            t2 = t * t
            nu = t * (-_C2 * t2 - _C1)
            z_v[zb, r * 4 + rr, sl] = t / (1.0 + jnp.exp(nu))
        return cr

      lax.fori_loop(0, _CH // 4, rowfn, 0)
      issue_scatter(j, zb)

    def sixpack(m, carry):
      for p in range(6):
        emit_chunk(m * 6 + p, p)
      return carry

    lax.fori_loop(0, _CHUNKS // 6, sixpack, 0)
    wait_scatter((_CHUNKS - 2) % 3)
    wait_scatter((_CHUNKS - 1) % 3)
    plsc.subcore_barrier()
    pltpu.sync_copy(agg_sp.at[pl.ds(row0, _RPS)],
                    agg_out.at[c, pl.ds(row0, _RPS)])

  return pl.kernel(body,
                   out_type=jax.ShapeDtypeStruct((_NC, _NPAD, _H), jnp.float32),
                   mesh=mesh, scratch_types=tuple(scratch))


def _make_sc_cnt():
  mesh = plsc.VectorSubcoreMesh(core_axis_name="c", subcore_axis_name="s")
  scratch = [
      pltpu.VMEM((_CHUNKS, _CH), jnp.int32),
      pltpu.VMEM((_CH, 16), jnp.float32),
      pltpu.VMEM_SHARED((_NPAD, 16), jnp.float32),
  ]

  def body(dst_i, zer16, one16, cnt_out, dst_v, one_v, cnt_sp):
    c = lax.axis_index("c")
    s = lax.axis_index("s")
    w = s * _NC + c
    row0 = s * _RPS
    pltpu.sync_copy(zer16.at[pl.ds(row0, _RPS)], cnt_sp.at[pl.ds(row0, _RPS)])
    pltpu.sync_copy(dst_i.at[pl.ds(w * _CHUNKS, _CHUNKS)], dst_v)
    pltpu.sync_copy(one16, one_v)
    plsc.subcore_barrier()

    def chunk(j, carry):
      pltpu.sync_copy(one_v, cnt_sp.at[dst_v.at[j]], add=True)
      return carry

    lax.fori_loop(0, _CHUNKS, chunk, 0)
    plsc.subcore_barrier()
    pltpu.sync_copy(cnt_sp.at[pl.ds(row0, _RPS)],
                    cnt_out.at[c, pl.ds(row0, _RPS)])

  return pl.kernel(body,
                   out_type=jax.ShapeDtypeStruct((_NC, _NPAD, 16), jnp.float32),
                   mesh=mesh, scratch_types=tuple(scratch))


_sc_agg = _make_sc_agg()
_sc_cnt = _make_sc_cnt()


def _prep_body(nf, w_in, b_in, wm1x0, x1, y1):
  x = jax.nn.gelu(
      jnp.dot(nf[...], w_in[...], preferred_element_type=jnp.float32,
              precision=lax.Precision.HIGHEST)
      + b_in[...])
  x1[...] = x
  y1[...] = jnp.dot(x, wm1x0[...], preferred_element_type=jnp.float32,
              precision=lax.Precision.HIGHEST)


_prep = pl.pallas_call(
    _prep_body,
    grid=(_GN,),
    in_specs=[
        pl.BlockSpec((_BN, _H), lambda i: (i, 0)),
        pl.BlockSpec((_H, _H), lambda i: (0, 0)),
        pl.BlockSpec((1, _H), lambda i: (0, 0)),
        pl.BlockSpec((_H, _H), lambda i: (0, 0)),
    ],
    out_specs=[pl.BlockSpec((_BN, _H), lambda i: (i, 0))] * 2,
    out_shape=[jax.ShapeDtypeStruct((_NPAD, _H), jnp.float32)] * 2,
)


def _z_body(ef, wc, bc, z0, z1, z2):
  t = jnp.dot(ef[...], wc[...], preferred_element_type=jnp.float32,
              precision=lax.Precision.HIGHEST) + bc[...]
  z0[...] = t[:, :_H]
  z1[...] = t[:, _H:2 * _H]
  z2[...] = t[:, 2 * _H:]


_zcalc = pl.pallas_call(
    _z_body,
    grid=(_EPAD // _BZ,),
    in_specs=[
        pl.BlockSpec((_BZ, _DE), lambda i: (i, 0)),
        pl.BlockSpec((_DE, 3 * _H), lambda i: (0, 0)),
        pl.BlockSpec((1, 3 * _H), lambda i: (0, 0)),
    ],
    out_specs=[pl.BlockSpec((_BZ, _H), lambda i: (i, 0))] * 3,
    out_shape=[jax.ShapeDtypeStruct((_EPAD, _H), jnp.float32)] * 3,
)


def _update_core(x, agg, cnt, wux, wua, wm2, bm2, bu, lns, lnb):
  pre = agg[0] + agg[1]
  cv = cnt[0, :, :1] + cnt[1, :, :1]
  w2u = jnp.dot(wm2[...], wua[...], preferred_element_type=jnp.float32,
              precision=lax.Precision.HIGHEST)
  cvec = jnp.dot(bm2[...], wua[...], preferred_element_type=jnp.float32,
              precision=lax.Precision.HIGHEST)
  h = (jnp.dot(x[...], wux[...], preferred_element_type=jnp.float32,
              precision=lax.Precision.HIGHEST)
       + jnp.dot(pre, w2u, preferred_element_type=jnp.float32,
              precision=lax.Precision.HIGHEST)
       + cv * cvec + bu[...])
  m = jnp.mean(h, axis=-1, keepdims=True)
  v = jnp.mean(jnp.square(h - m), axis=-1, keepdims=True)
  hn = (h - m) / jnp.sqrt(v + 1e-6) * lns[...] + lnb[...]
  return jax.nn.gelu(hn) + x[...]


def _upd_mid_body(x, agg, cnt, wux, wua, wm2, bm2, bu, lns, lnb, wm1xn,
                  xn_out, yn_out):
  xn = _update_core(x, agg, cnt, wux, wua, wm2, bm2, bu, lns, lnb)
  xn_out[...] = xn
  yn_out[...] = jnp.dot(xn, wm1xn[...], preferred_element_type=jnp.float32,
              precision=lax.Precision.HIGHEST)


def _upd_last_body(x, agg, cnt, wux, wua, wm2, bm2, bu, lns, lnb,
                   wg1, bg1, wg2, bg2, wot, bo, out, acc):
  i = pl.program_id(0)
  xn = _update_core(x, agg, cnt, wux, wua, wm2, bm2, bu, lns, lnb)
  rid = lax.broadcasted_iota(jnp.int32, (_BN, 1), 0) + i * _BN
  part = jnp.sum(jnp.where(rid < _N, xn, 0.0), axis=0, keepdims=True)

  @pl.when(i == 0)
  def _():
    acc[...] = jnp.zeros_like(acc)

  acc[...] += part

  @pl.when(i == _GN - 1)
  def _():
    g = acc[...]
    g1 = jax.nn.gelu(
        jnp.dot(g, wg1[...], preferred_element_type=jnp.float32,
              precision=lax.Precision.HIGHEST) + bg1[...])
    g2 = jax.nn.gelu(
        jnp.dot(g1, wg2[...], preferred_element_type=jnp.float32,
              precision=lax.Precision.HIGHEST) + bg2[...])
    out[...] = jnp.sum(g2 * wot[...], axis=-1, keepdims=True) + bo[...]


_spec_x = pl.BlockSpec((_BN, _H), lambda i: (i, 0))
_spec_agg = pl.BlockSpec((_NC, _BN, _H), lambda i: (0, i, 0))
_spec_cnt = pl.BlockSpec((_NC, _BN, 16), lambda i: (0, i, 0))
_spec_w = pl.BlockSpec((_H, _H), lambda i: (0, 0))
_spec_b = pl.BlockSpec((1, _H), lambda i: (0, 0))

_upd_mid = pl.pallas_call(
    _upd_mid_body,
    grid=(_GN,),
    in_specs=[_spec_x, _spec_agg, _spec_cnt, _spec_w, _spec_w, _spec_w,
              _spec_b, _spec_b, _spec_b, _spec_b, _spec_w],
    out_specs=[_spec_x, _spec_x],
    out_shape=[jax.ShapeDtypeStruct((_NPAD, _H), jnp.float32)] * 2,
)

_upd_last = pl.pallas_call(
    _upd_last_body,
    grid=(_GN,),
    in_specs=[_spec_x, _spec_agg, _spec_cnt, _spec_w, _spec_w, _spec_w,
              _spec_b, _spec_b, _spec_b, _spec_b,
              pl.BlockSpec((_H, 2 * _H), lambda i: (0, 0)),
              pl.BlockSpec((1, 2 * _H), lambda i: (0, 0)),
              pl.BlockSpec((2 * _H, _H), lambda i: (0, 0)),
              _spec_b, _spec_b,
              pl.BlockSpec((1, 1), lambda i: (0, 0))],
    out_specs=pl.BlockSpec((1, 1), lambda i: (0, 0)),
    out_shape=jax.ShapeDtypeStruct((1, 1), jnp.float32),
    scratch_shapes=[pltpu.VMEM((1, _H), jnp.float32)],
)


def kernel(node_features, edge_index, edge_features, params):
  p = params
  nf = jnp.pad(node_features, ((0, _NPAD - _N), (0, 0)))
  order = jnp.argsort(edge_index[1])
  src_s = edge_index[0][order]
  dst_s = edge_index[1][order]
  ef_s = edge_features[order]
  src = jnp.pad(src_s, (0, _EPAD - _E)).reshape(_EPAD // _CH, _CH)
  dst = jnp.pad(dst_s, (0, _EPAD - _E),
                constant_values=_N).reshape(_EPAD // _CH, _CH)
  ef = jnp.pad(ef_s, ((0, _EPAD - _E), (0, 0)))
  layers = p['layers']
  wm1x = [lp['Wm1'][:_H] for lp in layers]
  wm1e_cat = jnp.concatenate([lp['Wm1'][_H:] for lp in layers], axis=1)
  bm1_cat = jnp.concatenate([lp['bm1'] for lp in layers])[None]
  zer = jnp.zeros((_NPAD, _H), jnp.float32)
  zer16 = jnp.zeros((_NPAD, 16), jnp.float32)
  one16 = jnp.ones((_CH, 16), jnp.float32)

  x, y = _prep(nf, p['W_in'], p['b_in'][None], wm1x[0])
  zs = _zcalc(ef, wm1e_cat, bm1_cat)

  cnt = _sc_cnt(dst, zer16, one16)
  out = None
  for l, lp in enumerate(layers):
    agg = _sc_agg(y, zs[l], src, dst, zer)
    wux, wua = lp['Wu'][:_H], lp['Wu'][_H:]
    common = (x, agg, cnt, wux, wua, lp['Wm2'], lp['bm2'][None],
              lp['bu'][None], lp['ln_s'][None], lp['ln_b'][None])
    if l < _L - 1:
      x, y = _upd_mid(*common, wm1x[l + 1])
    else:
      out = _upd_last(*common, p['Wg1'], p['bg1'][None], p['Wg2'],
                      p['bg2'][None], p['Wo'].T, p['bo'][None])
  return out[0, 0]


# speedup vs baseline: 1.5361x; 1.5361x over previous
import functools

import jax
import jax.numpy as jnp
from jax import lax
from jax.experimental import pallas as pl
from jax.experimental.pallas import tpu as pltpu
from jax.experimental.pallas import tpu_sc as plsc

_N, _E, _H, _DE, _L = 10000, 320000, 128, 16, 3
_NC, _NS = 2, 16
_NW = _NC * _NS
_CH = 128
_CHUNKS = 80
_IG = 8
_EPT = _CHUNKS * _CH
_EPAD = _EPT * _NW
_NPAD = 10112
_RPS = _NPAD // _NS
_BN = 1264
_GN = _NPAD // _BN
_BZ = 4096
_C1 = 1.5957691216057308
_C2 = _C1 * 0.044715


def _make_sc_agg():
  mesh = plsc.VectorSubcoreMesh(core_axis_name="c", subcore_axis_name="s")
  scratch = [
      pltpu.VMEM((_IG, _CH), jnp.int32),
      pltpu.VMEM((_IG, _CH), jnp.int32),
      pltpu.VMEM((_CH, _H), jnp.float32),
      pltpu.VMEM((_CH, _H), jnp.float32),
      pltpu.VMEM_SHARED((_NPAD, _H), jnp.float32),
      pltpu.SemaphoreType.DMA,
      pltpu.SemaphoreType.DMA,
  ]

  def body(y, z, src_i, dst_i, zer, agg_out,
           src_v, dst_v, g_v, z_v, agg_sp, sem_g, sem_z):
    c = lax.axis_index("c")
    s = lax.axis_index("s")
    w = s * _NC + c
    row0 = s * _RPS
    irow0 = w * _CHUNKS

    pltpu.sync_copy(zer.at[pl.ds(row0, _RPS)], agg_sp.at[pl.ds(row0, _RPS)])
    plsc.subcore_barrier()

    def group(jg, carry):
      pltpu.sync_copy(src_i.at[pl.ds(irow0 + jg * _IG, _IG)], src_v)
      pltpu.sync_copy(dst_i.at[pl.ds(irow0 + jg * _IG, _IG)], dst_v)

      def chunk(jj, carry2):
        j = jg * _IG + jj
        gd = pltpu.async_copy(y.at[src_v.at[jj]], g_v, sem_g)
        zd = pltpu.async_copy(z.at[pl.ds(w * _EPT + j * _CH, _CH)], z_v,
                              sem_z)
        gd.wait()
        zd.wait()

        @plsc.parallel_loop(0, _CH, unroll=4)
        def _(r):
          for cc in range(_H // 16):
            sl = pl.ds(cc * 16, 16)
            t = g_v[r, sl] + z_v[r, sl]
            t2 = t * t
            nu = t * (-_C2 * t2 - _C1)
            g_v[r, sl] = t / (1.0 + jnp.exp(nu))

        pltpu.sync_copy(g_v, agg_sp.at[dst_v.at[jj]], add=True)
        return carry2

      lax.fori_loop(0, _IG, chunk, 0)
      return carry

    lax.fori_loop(0, _CHUNKS // _IG, group, 0)
    plsc.subcore_barrier()
    pltpu.sync_copy(agg_sp.at[pl.ds(row0, _RPS)],
                    agg_out.at[c, pl.ds(row0, _RPS)])

  return pl.kernel(body,
                   out_type=jax.ShapeDtypeStruct((_NC, _NPAD, _H), jnp.float32),
                   mesh=mesh, scratch_types=tuple(scratch))


def _make_sc_cnt():
  mesh = plsc.VectorSubcoreMesh(core_axis_name="c", subcore_axis_name="s")
  scratch = [
      pltpu.VMEM((_CHUNKS, _CH), jnp.int32),
      pltpu.VMEM((_CH, 16), jnp.float32),
      pltpu.VMEM_SHARED((_NPAD, 16), jnp.float32),
  ]

  def body(dst_i, zer16, one16, cnt_out, dst_v, one_v, cnt_sp):
    c = lax.axis_index("c")
    s = lax.axis_index("s")
    w = s * _NC + c
    row0 = s * _RPS
    pltpu.sync_copy(zer16.at[pl.ds(row0, _RPS)], cnt_sp.at[pl.ds(row0, _RPS)])
    pltpu.sync_copy(dst_i.at[pl.ds(w * _CHUNKS, _CHUNKS)], dst_v)
    pltpu.sync_copy(one16, one_v)
    plsc.subcore_barrier()

    def chunk(j, carry):
      pltpu.sync_copy(one_v, cnt_sp.at[dst_v.at[j]], add=True)
      return carry

    lax.fori_loop(0, _CHUNKS, chunk, 0)
    plsc.subcore_barrier()
    pltpu.sync_copy(cnt_sp.at[pl.ds(row0, _RPS)],
                    cnt_out.at[c, pl.ds(row0, _RPS)])

  return pl.kernel(body,
                   out_type=jax.ShapeDtypeStruct((_NC, _NPAD, 16), jnp.float32),
                   mesh=mesh, scratch_types=tuple(scratch))


_sc_agg = _make_sc_agg()
_sc_cnt = _make_sc_cnt()


def _prep_body(nf, w_in, b_in, wm1x0, x1, y1):
  x = jax.nn.gelu(
      jnp.dot(nf[...], w_in[...], preferred_element_type=jnp.float32,
              precision=lax.Precision.HIGHEST)
      + b_in[...])
  x1[...] = x
  y1[...] = jnp.dot(x, wm1x0[...], preferred_element_type=jnp.float32,
              precision=lax.Precision.HIGHEST)


_prep = pl.pallas_call(
    _prep_body,
    grid=(_GN,),
    in_specs=[
        pl.BlockSpec((_BN, _H), lambda i: (i, 0)),
        pl.BlockSpec((_H, _H), lambda i: (0, 0)),
        pl.BlockSpec((1, _H), lambda i: (0, 0)),
        pl.BlockSpec((_H, _H), lambda i: (0, 0)),
    ],
    out_specs=[pl.BlockSpec((_BN, _H), lambda i: (i, 0))] * 2,
    out_shape=[jax.ShapeDtypeStruct((_NPAD, _H), jnp.float32)] * 2,
)


def _z_body(ef, wc, bc, z0, z1, z2):
  t = jnp.dot(ef[...], wc[...], preferred_element_type=jnp.float32,
              precision=lax.Precision.HIGHEST) + bc[...]
  z0[...] = t[:, :_H]
  z1[...] = t[:, _H:2 * _H]
  z2[...] = t[:, 2 * _H:]


_zcalc = pl.pallas_call(
    _z_body,
    grid=(_EPAD // _BZ,),
    in_specs=[
        pl.BlockSpec((_BZ, _DE), lambda i: (i, 0)),
        pl.BlockSpec((_DE, 3 * _H), lambda i: (0, 0)),
        pl.BlockSpec((1, 3 * _H), lambda i: (0, 0)),
    ],
    out_specs=[pl.BlockSpec((_BZ, _H), lambda i: (i, 0))] * 3,
    out_shape=[jax.ShapeDtypeStruct((_EPAD, _H), jnp.float32)] * 3,
)


def _update_core(x, agg, cnt, wux, wua, wm2, bm2, bu, lns, lnb):
  pre = agg[0] + agg[1]
  cv = cnt[0, :, :1] + cnt[1, :, :1]
  w2u = jnp.dot(wm2[...], wua[...], preferred_element_type=jnp.float32,
              precision=lax.Precision.HIGHEST)
  cvec = jnp.dot(bm2[...], wua[...], preferred_element_type=jnp.float32,
              precision=lax.Precision.HIGHEST)
  h = (jnp.dot(x[...], wux[...], preferred_element_type=jnp.float32,
              precision=lax.Precision.HIGHEST)
       + jnp.dot(pre, w2u, preferred_element_type=jnp.float32,
              precision=lax.Precision.HIGHEST)
       + cv * cvec + bu[...])
  m = jnp.mean(h, axis=-1, keepdims=True)
  v = jnp.mean(jnp.square(h - m), axis=-1, keepdims=True)
  hn = (h - m) / jnp.sqrt(v + 1e-6) * lns[...] + lnb[...]
  return jax.nn.gelu(hn) + x[...]


def _upd_mid_body(x, agg, cnt, wux, wua, wm2, bm2, bu, lns, lnb, wm1xn,
                  xn_out, yn_out):
  xn = _update_core(x, agg, cnt, wux, wua, wm2, bm2, bu, lns, lnb)
  xn_out[...] = xn
  yn_out[...] = jnp.dot(xn, wm1xn[...], preferred_element_type=jnp.float32,
              precision=lax.Precision.HIGHEST)


def _upd_last_body(x, agg, cnt, wux, wua, wm2, bm2, bu, lns, lnb,
                   wg1, bg1, wg2, bg2, wot, bo, out, acc):
  i = pl.program_id(0)
  xn = _update_core(x, agg, cnt, wux, wua, wm2, bm2, bu, lns, lnb)
  rid = lax.broadcasted_iota(jnp.int32, (_BN, 1), 0) + i * _BN
  part = jnp.sum(jnp.where(rid < _N, xn, 0.0), axis=0, keepdims=True)

  @pl.when(i == 0)
  def _():
    acc[...] = jnp.zeros_like(acc)

  acc[...] += part

  @pl.when(i == _GN - 1)
  def _():
    g = acc[...]
    g1 = jax.nn.gelu(
        jnp.dot(g, wg1[...], preferred_element_type=jnp.float32,
              precision=lax.Precision.HIGHEST) + bg1[...])
    g2 = jax.nn.gelu(
        jnp.dot(g1, wg2[...], preferred_element_type=jnp.float32,
              precision=lax.Precision.HIGHEST) + bg2[...])
    out[...] = jnp.sum(g2 * wot[...], axis=-1, keepdims=True) + bo[...]


_spec_x = pl.BlockSpec((_BN, _H), lambda i: (i, 0))
_spec_agg = pl.BlockSpec((_NC, _BN, _H), lambda i: (0, i, 0))
_spec_cnt = pl.BlockSpec((_NC, _BN, 16), lambda i: (0, i, 0))
_spec_w = pl.BlockSpec((_H, _H), lambda i: (0, 0))
_spec_b = pl.BlockSpec((1, _H), lambda i: (0, 0))

_upd_mid = pl.pallas_call(
    _upd_mid_body,
    grid=(_GN,),
    in_specs=[_spec_x, _spec_agg, _spec_cnt, _spec_w, _spec_w, _spec_w,
              _spec_b, _spec_b, _spec_b, _spec_b, _spec_w],
    out_specs=[_spec_x, _spec_x],
    out_shape=[jax.ShapeDtypeStruct((_NPAD, _H), jnp.float32)] * 2,
)

_upd_last = pl.pallas_call(
    _upd_last_body,
    grid=(_GN,),
    in_specs=[_spec_x, _spec_agg, _spec_cnt, _spec_w, _spec_w, _spec_w,
              _spec_b, _spec_b, _spec_b, _spec_b,
              pl.BlockSpec((_H, 2 * _H), lambda i: (0, 0)),
              pl.BlockSpec((1, 2 * _H), lambda i: (0, 0)),
              pl.BlockSpec((2 * _H, _H), lambda i: (0, 0)),
              _spec_b, _spec_b,
              pl.BlockSpec((1, 1), lambda i: (0, 0))],
    out_specs=pl.BlockSpec((1, 1), lambda i: (0, 0)),
    out_shape=jax.ShapeDtypeStruct((1, 1), jnp.float32),
    scratch_shapes=[pltpu.VMEM((1, _H), jnp.float32)],
)


def kernel(node_features, edge_index, edge_features, params):
  p = params
  nf = jnp.pad(node_features, ((0, _NPAD - _N), (0, 0)))
  src = jnp.pad(edge_index[0], (0, _EPAD - _E)).reshape(_EPAD // _CH, _CH)
  dst = jnp.pad(edge_index[1], (0, _EPAD - _E),
                constant_values=_N).reshape(_EPAD // _CH, _CH)
  ef = jnp.pad(edge_features, ((0, _EPAD - _E), (0, 0)))
  layers = p['layers']
  wm1x = [lp['Wm1'][:_H] for lp in layers]
  wm1e_cat = jnp.concatenate([lp['Wm1'][_H:] for lp in layers], axis=1)
  bm1_cat = jnp.concatenate([lp['bm1'] for lp in layers])[None]
  zer = jnp.zeros((_NPAD, _H), jnp.float32)
  zer16 = jnp.zeros((_NPAD, 16), jnp.float32)
  one16 = jnp.ones((_CH, 16), jnp.float32)

  x, y = _prep(nf, p['W_in'], p['b_in'][None], wm1x[0])
  zs = _zcalc(ef, wm1e_cat, bm1_cat)

  cnt = _sc_cnt(dst, zer16, one16)
  out = None
  for l, lp in enumerate(layers):
    agg = _sc_agg(y, zs[l], src, dst, zer)
    wux, wua = lp['Wu'][:_H], lp['Wu'][_H:]
    common = (x, agg, cnt, wux, wua, lp['Wm2'], lp['bm2'][None],
              lp['bu'][None], lp['ln_s'][None], lp['ln_b'][None])
    if l < _L - 1:
      x, y = _upd_mid(*common, wm1x[l + 1])
    else:
      out = _upd_last(*common, p['Wg1'], p['bg1'][None], p['Wg2'],
                      p['bg2'][None], p['Wo'].T, p['bo'][None])
  return out[0, 0]
